# baseline (speedup 1.0000x reference)
import functools

import jax
import jax.numpy as jnp
from jax import lax
from jax.experimental import pallas as pl
from jax.experimental.pallas import tpu as pltpu
from jax.experimental.pallas import tpu_sc as plsc

_NC = 2
_NS = 16
_C = 80


def _mish(y):
    sp = jnp.maximum(y, 0.0) + jnp.log1p(jnp.exp(-jnp.abs(y)))
    return y * jnp.tanh(sp)


def _dense_mish_body(x_ref, w_ref, b_ref, o_ref):
    y = jnp.dot(x_ref[...].astype(jnp.bfloat16),
                w_ref[...].astype(jnp.bfloat16),
                preferred_element_type=jnp.float32)
    o_ref[...] = _mish(y + b_ref[...])


def _dense_mish(x, W, b, blk):
    R, D = x.shape
    return pl.pallas_call(
        _dense_mish_body,
        grid=(R // blk,),
        in_specs=[
            pl.BlockSpec((blk, D), lambda i: (i, 0)),
            pl.BlockSpec((D, D), lambda i: (0, 0)),
            pl.BlockSpec((1, D), lambda i: (0, 0)),
        ],
        out_specs=pl.BlockSpec((blk, D), lambda i: (i, 0)),
        out_shape=jax.ShapeDtypeStruct((R, D), jnp.float32),
    )(x, W, b.reshape(1, D))


def _proj_body(hx_ref, p0_ref, w_ref, b_ref, o_ref):
    acc = hx_ref[...] + p0_ref[...]
    y = jnp.dot(acc.astype(jnp.bfloat16), w_ref[...].astype(jnp.bfloat16),
                preferred_element_type=jnp.float32)
    o_ref[...] = _mish(y + b_ref[...])


def _proj(hx, p0, W, b, blk):
    R, D = hx.shape
    return pl.pallas_call(
        _proj_body,
        grid=(R // blk,),
        in_specs=[
            pl.BlockSpec((blk, D), lambda i: (i, 0)),
            pl.BlockSpec((blk, D), lambda i: (i, 0)),
            pl.BlockSpec((D, D), lambda i: (0, 0)),
            pl.BlockSpec((1, D), lambda i: (0, 0)),
        ],
        out_specs=pl.BlockSpec((blk, D), lambda i: (i, 0)),
        out_shape=jax.ShapeDtypeStruct((R, D), jnp.float32),
    )(hx, p0, W, b.reshape(1, D))


_W = 6
_HALF = 5120
_TRASH = 128
_ACC_R = _HALF + _TRASH
_RPT_Z = _ACC_R // _NS
_RPT_O = _HALF // _NS


def _pack_dst_flags(dst):
    s1 = jnp.zeros(dst.shape, jnp.bool_)
    for dlt in range(1, _W + 1):
        s1 = s1 | (dst == jnp.roll(dst, -dlt))
    s2 = jnp.zeros(dst.shape, jnp.bool_)
    for dlt in range(1, _W + 1):
        s2 = s2 | ((dst == jnp.roll(dst, -dlt)) & jnp.roll(s1, -dlt))
    return dst * 4 + s1.astype(jnp.int32) + 2 * s2.astype(jnp.int32)


def _sc_agg(hx, he, src, dstf):
    N, D = hx.shape
    E = he.shape[0]
    ept = E // _NS
    nchunk = ept // _C

    mesh = plsc.VectorSubcoreMesh(core_axis_name="c", subcore_axis_name="s",
                                  num_cores=2)

    def body(hx_hbm, he_hbm, src_hbm, dstf_hbm, out_hbm,
             src_v, da_v, db_v, dc_v, dx_v, m_v, g_v, z_v, acc_sh):
        c = lax.axis_index("c")
        s = lax.axis_index("s")
        node_base = c * _HALF

        def zrow(r, carry):
            for k in range(8):
                z_v[r, pl.ds(k * 16, 16)] = jnp.zeros((16,), jnp.float32)
            return carry
        lax.fori_loop(0, _RPT_Z, zrow, 0)
        pltpu.sync_copy(z_v, acc_sh.at[pl.ds(s * _RPT_Z, _RPT_Z)])
        plsc.subcore_barrier()

        def chunk(j, carry):
            base = s * ept + j * _C
            pltpu.sync_copy(src_hbm.at[pl.ds(base, _C)], src_v)
            pltpu.sync_copy(dstf_hbm.at[pl.ds(base, _C)], dx_v)
            pltpu.sync_copy(he_hbm.at[pl.ds(base, _C)], m_v)
            pltpu.sync_copy(hx_hbm.at[src_v], g_v)

            for q in range(_C // 16):
                sl = pl.ds(q * 16, 16)
                v = dx_v[sl]
                s1g = v & 1
                s2g = lax.shift_right_logical(v, 1) & 1
                dval = lax.shift_right_logical(v, 2)
                local = dval - node_base
                inr = (local >= 0) & (local < _HALF)
                trash = lax.iota(jnp.int32, 16) + (_HALF + q * 16)
                da_v[sl] = jnp.where(inr & (s1g < 1), local, trash)
                db_v[sl] = jnp.where(inr & (s1g > 0) & (s2g < 1),
                                     local, trash)
                dc_v[sl] = jnp.where(inr & (s2g > 0), local, trash)

            def rb(r, cc):
                for k in range(8):
                    slk = pl.ds(k * 16, 16)
                    m_v[r, slk] = jnp.maximum(m_v[r, slk] + g_v[r, slk],
                                              0.0)
                return cc
            lax.fori_loop(0, _C, rb, 0)

            pltpu.sync_copy(m_v, acc_sh.at[da_v], add=True)
            pl.delay(300)
            pltpu.sync_copy(m_v, acc_sh.at[db_v], add=True)
            pl.delay(300)
            pltpu.sync_copy(m_v, acc_sh.at[dc_v], add=True)
            pl.delay(300)
            return carry
        lax.fori_loop(0, nchunk, chunk, 0)
        plsc.subcore_barrier()

        rows = pl.ds(s * _RPT_O, _RPT_O)
        orows = pl.ds(node_base + s * _RPT_O, _RPT_O)
        pltpu.sync_copy(acc_sh.at[rows], out_hbm.at[orows])

    f = pl.kernel(
        body,
        out_type=(),
        mesh=mesh,
        scratch_types=[
            pltpu.VMEM((_C,), jnp.int32),
            pltpu.VMEM((_C,), jnp.int32),
            pltpu.VMEM((_C,), jnp.int32),
            pltpu.VMEM((_C,), jnp.int32),
            pltpu.VMEM((_C,), jnp.int32),
            pltpu.VMEM((_C, D), jnp.float32),
            pltpu.VMEM((_C, D), jnp.float32),
            pltpu.VMEM((_RPT_Z, D), jnp.float32),
            pltpu.VMEM_SHARED((_ACC_R, D), jnp.float32),
        ],
    )
    out_ref = jax.new_ref(jnp.zeros((2 * _HALF, D), jnp.float32))
    f(hx, he, src, dstf, out_ref)
    return out_ref[...]


def _pool_body(gid_ref, h_ref, gamma_ref, beta_ref, wout_ref, bout_ref,
               o_ref, sums_ref, cnts_ref):
    i = pl.program_id(0)
    nsteps = pl.num_programs(0)
    G = sums_ref.shape[0]

    @pl.when(i == 0)
    def _():
        sums_ref[...] = jnp.zeros_like(sums_ref)
        cnts_ref[...] = jnp.zeros_like(cnts_ref)

    gid = gid_ref[...].reshape(1, -1)
    onehot = (lax.broadcasted_iota(jnp.int32, (G, gid.shape[1]), 0)
              == gid).astype(jnp.float32)
    sums_ref[...] += jnp.dot(onehot, h_ref[...],
                             preferred_element_type=jnp.float32,
                             precision=lax.Precision.HIGHEST)
    cnts_ref[...] += jnp.sum(onehot, axis=1, keepdims=True)

    @pl.when(i == nsteps - 1)
    def _():
        pooled = sums_ref[...] / jnp.maximum(cnts_ref[...], 1.0)
        mu = jnp.mean(pooled, axis=0, keepdims=True)
        var = jnp.mean((pooled - mu) ** 2, axis=0, keepdims=True)
        xn = (pooled - mu) * lax.rsqrt(var + 1e-5)
        xn = xn * gamma_ref[...] + beta_ref[...]
        o_ref[...] = jnp.dot(xn.astype(jnp.bfloat16),
                             wout_ref[...].astype(jnp.bfloat16),
                             preferred_element_type=jnp.float32) + bout_ref[...]


def _pool_bn_head(h, graph_ids, gamma, beta, W_out, b_out, G, blk):
    N, D = h.shape
    nsteps = N // blk
    gid3 = graph_ids.reshape(nsteps, 1, blk)
    return pl.pallas_call(
        _pool_body,
        grid=(nsteps,),
        in_specs=[
            pl.BlockSpec((1, 1, blk), lambda i: (i, 0, 0)),
            pl.BlockSpec((blk, D), lambda i: (i, 0)),
            pl.BlockSpec((1, D), lambda i: (0, 0)),
            pl.BlockSpec((1, D), lambda i: (0, 0)),
            pl.BlockSpec((D, 1), lambda i: (0, 0)),
            pl.BlockSpec((1, 1), lambda i: (0, 0)),
        ],
        out_specs=pl.BlockSpec((G, 1), lambda i: (0, 0)),
        out_shape=jax.ShapeDtypeStruct((G, 1), jnp.float32),
        scratch_shapes=[
            pltpu.VMEM((G, D), jnp.float32),
            pltpu.VMEM((G, 1), jnp.float32),
        ],
    )(gid3, h, gamma.reshape(1, D), beta.reshape(1, D),
      W_out, b_out.reshape(1, 1))


def kernel(x, edge_attr, edge_index, graph_ids,
           W_d0, b_d0, W_e0, b_e0, W_p0, b_p0,
           W_d1, b_d1, W_e1, b_e1, W_p1, b_p1,
           gamma, beta, W_out, b_out):
    N, D = x.shape
    E = edge_attr.shape[0]
    G = 256
    src = edge_index[0]
    dst = edge_index[1]

    dstf = _pack_dst_flags(dst)
    h = x
    for (Wd, bd, We, be, Wp, bp) in (
            (W_d0, b_d0, W_e0, b_e0, W_p0, b_p0),
            (W_d1, b_d1, W_e1, b_e1, W_p1, b_p1)):
        hx = _dense_mish(h, Wd, bd, blk=2000)
        he = _dense_mish(edge_attr, We, be, blk=2000)
        p0 = _sc_agg(hx, he, src, dstf)
        h = _proj(hx, p0, Wp, bp, blk=2000)

    return _pool_bn_head(h, graph_ids, gamma, beta, W_out, b_out, G, blk=1000)


# speedup vs baseline: 2.0716x; 2.0716x over previous
import functools

import jax
import jax.numpy as jnp
from jax import lax
from jax.experimental import pallas as pl
from jax.experimental.pallas import tpu as pltpu
from jax.experimental.pallas import tpu_sc as plsc

_NC = 2
_NS = 16
_C = 80


def _mish(y):
    sp = jnp.maximum(y, 0.0) + jnp.log1p(jnp.exp(-jnp.abs(y)))
    return y * jnp.tanh(sp)


def _dense_mish_body(x_ref, w_ref, b_ref, o_ref):
    y = jnp.dot(x_ref[...].astype(jnp.bfloat16),
                w_ref[...].astype(jnp.bfloat16),
                preferred_element_type=jnp.float32)
    o_ref[...] = _mish(y + b_ref[...])


def _dense_mish(x, W, b, blk):
    R, D = x.shape
    return pl.pallas_call(
        _dense_mish_body,
        grid=(R // blk,),
        in_specs=[
            pl.BlockSpec((blk, D), lambda i: (i, 0)),
            pl.BlockSpec((D, D), lambda i: (0, 0)),
            pl.BlockSpec((1, D), lambda i: (0, 0)),
        ],
        out_specs=pl.BlockSpec((blk, D), lambda i: (i, 0)),
        out_shape=jax.ShapeDtypeStruct((R, D), jnp.float32),
    )(x, W, b.reshape(1, D))


def _proj_body(hx_ref, p0_ref, w_ref, b_ref, o_ref):
    acc = hx_ref[...] + p0_ref[...]
    y = jnp.dot(acc.astype(jnp.bfloat16), w_ref[...].astype(jnp.bfloat16),
                preferred_element_type=jnp.float32)
    o_ref[...] = _mish(y + b_ref[...])


def _proj(hx, p0, W, b, blk):
    R, D = hx.shape
    return pl.pallas_call(
        _proj_body,
        grid=(R // blk,),
        in_specs=[
            pl.BlockSpec((blk, D), lambda i: (i, 0)),
            pl.BlockSpec((blk, D), lambda i: (i, 0)),
            pl.BlockSpec((D, D), lambda i: (0, 0)),
            pl.BlockSpec((1, D), lambda i: (0, 0)),
        ],
        out_specs=pl.BlockSpec((blk, D), lambda i: (i, 0)),
        out_shape=jax.ShapeDtypeStruct((R, D), jnp.float32),
    )(hx, p0, W, b.reshape(1, D))


_W = 6
_HALF = 5120
_TRASH = 128
_ACC_R = _HALF + _TRASH
_RPT_Z = _ACC_R // _NS
_RPT_O = _HALF // _NS


def _pack_dst_flags(dst):
    s1 = jnp.zeros(dst.shape, jnp.bool_)
    for dlt in range(1, _W + 1):
        s1 = s1 | (dst == jnp.roll(dst, -dlt))
    s2 = jnp.zeros(dst.shape, jnp.bool_)
    for dlt in range(1, _W + 1):
        s2 = s2 | ((dst == jnp.roll(dst, -dlt)) & jnp.roll(s1, -dlt))
    return dst * 4 + s1.astype(jnp.int32) + 2 * s2.astype(jnp.int32)


def _sc_agg(hx, he, src, dstf):
    N, D = hx.shape
    E = he.shape[0]
    ept = E // _NS
    nchunk = ept // _C

    mesh = plsc.VectorSubcoreMesh(core_axis_name="c", subcore_axis_name="s",
                                  num_cores=2)

    def body(hx_hbm, he_hbm, src_hbm, dstf_hbm, out_hbm,
             src_v, da_v, db_v, dc_v, dx_v, m_v, g_v, z_v, acc_sh):
        c = lax.axis_index("c")
        s = lax.axis_index("s")
        node_base = c * _HALF

        def zrow(r, carry):
            for k in range(8):
                z_v[r, pl.ds(k * 16, 16)] = jnp.zeros((16,), jnp.float32)
            return carry
        lax.fori_loop(0, _RPT_Z, zrow, 0)
        pltpu.sync_copy(z_v, acc_sh.at[pl.ds(s * _RPT_Z, _RPT_Z)])
        plsc.subcore_barrier()

        def chunk(j, carry):
            base = s * ept + j * _C
            pltpu.sync_copy(src_hbm.at[pl.ds(base, _C)], src_v)
            pltpu.sync_copy(dstf_hbm.at[pl.ds(base, _C)], dx_v)
            pltpu.sync_copy(he_hbm.at[pl.ds(base, _C)], m_v)
            pltpu.sync_copy(hx_hbm.at[src_v], g_v)

            for q in range(_C // 16):
                sl = pl.ds(q * 16, 16)
                dval = lax.shift_right_logical(dx_v[sl], 2)
                local = dval - node_base
                inr = (local >= 0) & (local < _HALF)
                trash = lax.iota(jnp.int32, 16) + (_HALF + q * 16)
                da_v[sl] = jnp.where(inr, local, trash)

            def rb(r, cc):
                for k in range(8):
                    slk = pl.ds(k * 16, 16)
                    m_v[r, slk] = jnp.maximum(m_v[r, slk] + g_v[r, slk],
                                              0.0)
                return cc
            lax.fori_loop(0, _C, rb, 0)

            pltpu.sync_copy(m_v, acc_sh.at[da_v], add=True)
            return carry
        lax.fori_loop(0, nchunk, chunk, 0)
        plsc.subcore_barrier()

        rows = pl.ds(s * _RPT_O, _RPT_O)
        orows = pl.ds(node_base + s * _RPT_O, _RPT_O)
        pltpu.sync_copy(acc_sh.at[rows], out_hbm.at[orows])

    f = pl.kernel(
        body,
        out_type=(),
        mesh=mesh,
        scratch_types=[
            pltpu.VMEM((_C,), jnp.int32),
            pltpu.VMEM((_C,), jnp.int32),
            pltpu.VMEM((_C,), jnp.int32),
            pltpu.VMEM((_C,), jnp.int32),
            pltpu.VMEM((_C,), jnp.int32),
            pltpu.VMEM((_C, D), jnp.float32),
            pltpu.VMEM((_C, D), jnp.float32),
            pltpu.VMEM((_RPT_Z, D), jnp.float32),
            pltpu.VMEM_SHARED((_ACC_R, D), jnp.float32),
        ],
    )
    out_ref = jax.new_ref(jnp.zeros((2 * _HALF, D), jnp.float32))
    f(hx, he, src, dstf, out_ref)
    return out_ref[...]


def _pool_body(gid_ref, h_ref, gamma_ref, beta_ref, wout_ref, bout_ref,
               o_ref, sums_ref, cnts_ref):
    i = pl.program_id(0)
    nsteps = pl.num_programs(0)
    G = sums_ref.shape[0]

    @pl.when(i == 0)
    def _():
        sums_ref[...] = jnp.zeros_like(sums_ref)
        cnts_ref[...] = jnp.zeros_like(cnts_ref)

    gid = gid_ref[...].reshape(1, -1)
    onehot = (lax.broadcasted_iota(jnp.int32, (G, gid.shape[1]), 0)
              == gid).astype(jnp.float32)
    sums_ref[...] += jnp.dot(onehot, h_ref[...],
                             preferred_element_type=jnp.float32,
                             precision=lax.Precision.HIGHEST)
    cnts_ref[...] += jnp.sum(onehot, axis=1, keepdims=True)

    @pl.when(i == nsteps - 1)
    def _():
        pooled = sums_ref[...] / jnp.maximum(cnts_ref[...], 1.0)
        mu = jnp.mean(pooled, axis=0, keepdims=True)
        var = jnp.mean((pooled - mu) ** 2, axis=0, keepdims=True)
        xn = (pooled - mu) * lax.rsqrt(var + 1e-5)
        xn = xn * gamma_ref[...] + beta_ref[...]
        o_ref[...] = jnp.dot(xn.astype(jnp.bfloat16),
                             wout_ref[...].astype(jnp.bfloat16),
                             preferred_element_type=jnp.float32) + bout_ref[...]


def _pool_bn_head(h, graph_ids, gamma, beta, W_out, b_out, G, blk):
    N, D = h.shape
    nsteps = N // blk
    gid3 = graph_ids.reshape(nsteps, 1, blk)
    return pl.pallas_call(
        _pool_body,
        grid=(nsteps,),
        in_specs=[
            pl.BlockSpec((1, 1, blk), lambda i: (i, 0, 0)),
            pl.BlockSpec((blk, D), lambda i: (i, 0)),
            pl.BlockSpec((1, D), lambda i: (0, 0)),
            pl.BlockSpec((1, D), lambda i: (0, 0)),
            pl.BlockSpec((D, 1), lambda i: (0, 0)),
            pl.BlockSpec((1, 1), lambda i: (0, 0)),
        ],
        out_specs=pl.BlockSpec((G, 1), lambda i: (0, 0)),
        out_shape=jax.ShapeDtypeStruct((G, 1), jnp.float32),
        scratch_shapes=[
            pltpu.VMEM((G, D), jnp.float32),
            pltpu.VMEM((G, 1), jnp.float32),
        ],
    )(gid3, h, gamma.reshape(1, D), beta.reshape(1, D),
      W_out, b_out.reshape(1, 1))


def kernel(x, edge_attr, edge_index, graph_ids,
           W_d0, b_d0, W_e0, b_e0, W_p0, b_p0,
           W_d1, b_d1, W_e1, b_e1, W_p1, b_p1,
           gamma, beta, W_out, b_out):
    N, D = x.shape
    E = edge_attr.shape[0]
    G = 256
    src = edge_index[0]
    dst = edge_index[1]

    dstf = _pack_dst_flags(dst)
    h = x
    for (Wd, bd, We, be, Wp, bp) in (
            (W_d0, b_d0, W_e0, b_e0, W_p0, b_p0),
            (W_d1, b_d1, W_e1, b_e1, W_p1, b_p1)):
        hx = _dense_mish(h, Wd, bd, blk=2000)
        he = _dense_mish(edge_attr, We, be, blk=2000)
        p0 = _sc_agg(hx, he, src, dstf)
        h = _proj(hx, p0, Wp, bp, blk=2000)

    return _pool_bn_head(h, graph_ids, gamma, beta, W_out, b_out, G, blk=1000)
